# Initial kernel scaffold; baseline (speedup 1.0000x reference)
import jax
import jax.numpy as jnp
from jax.experimental import pallas as pl


def kernel(ab_pairs, vals, mask):
    raise NotImplementedError("write your pallas kernel here")


# speedup vs baseline: 1.4622x; 1.4622x over previous
import jax
import jax.numpy as jnp
from jax import lax
from jax.experimental import pallas as pl
from jax.experimental.pallas import tpu as pltpu
from jax.experimental.pallas import tpu_sc as plsc

BS, N, D, C = 8, 1024, 3, 128
M = 256
L = 16
NCH = N // L


def _wid():
    return lax.axis_index("s") * 2 + lax.axis_index("c")


def _fps_body(ab_ref, f0_ref, q_ref, rowbuf, dist_ref, chosen_ref, f0_v):
    w = _wid()

    @pl.when(w < BS)
    def _():
        b = w
        lanes = lax.iota(jnp.int32, L)
        pltpu.sync_copy(f0_ref, f0_v)
        f0 = jnp.max(jnp.where(lanes == b, f0_v[...].astype(jnp.float32),
                               0.0)).astype(jnp.int32)

        big = jnp.full((L,), 1e16, jnp.float32)
        for k in range(NCH):
            dist_ref[pl.ds(k * L, L)] = big
        idx_x = lanes * 3

        def outer(o, f):
            def inner(t, carry):
                f, ch = carry
                ch = jnp.where(lanes == t, f, ch)
                pltpu.sync_copy(ab_ref.at[b * N + f], rowbuf)
                bestv = jnp.full((L,), -1.0, jnp.float32)
                besti = jnp.zeros((L,), jnp.int32)
                for k in range(NCH):
                    base = k * 3 * L
                    x = plsc.load_gather(rowbuf, [idx_x + base])
                    y = plsc.load_gather(rowbuf, [idx_x + (base + 1)])
                    z = plsc.load_gather(rowbuf, [idx_x + (base + 2)])
                    dn = x * x + y * y + z * z
                    dm = jnp.minimum(dn, dist_ref[pl.ds(k * L, L)])
                    dist_ref[pl.ds(k * L, L)] = dm
                    upd = dm > bestv
                    bestv = jnp.where(upd, dm, bestv)
                    besti = jnp.where(upd, lanes + (k * L), besti)
                gm = jnp.max(bestv)
                cand = jnp.where(bestv == gm, besti.astype(jnp.float32), 1e9)
                return jnp.min(cand).astype(jnp.int32), ch

            f, ch = lax.fori_loop(0, L, inner, (f, jnp.zeros((L,), jnp.int32)))
            chosen_ref[pl.ds(pl.multiple_of(o * L, L), L)] = ch
            return f

        lax.fori_loop(0, M // L, outer, f0)
        pltpu.sync_copy(chosen_ref, q_ref.at[b])


def _gather_body(ab_flat, vals_h, mask_h, q_h, oab, ovals, omask,
                 q_v, q3_v, ridx_v, rows_v, stag, vidx_v, vrow,
                 mrow, msub, sem):
    w = _wid()
    b = w // 4
    i0_base = (w % 4) * 64
    lanes = lax.iota(jnp.int32, L)

    pltpu.sync_copy(q_h.at[b], q_v)
    for k in range(M // L):
        q3_v[pl.ds(k * L, L)] = q_v[pl.ds(k * L, L)] * 3

    for k in range(4):
        st = pl.multiple_of(i0_base + k * L, L)
        vidx_v[pl.ds(k * L, L)] = q_v[pl.ds(st, L)] + b * N
    pltpu.async_copy(vals_h.at[vidx_v], vrow, sem).wait()
    pltpu.sync_copy(vrow, ovals.at[b, pl.ds(pl.multiple_of(i0_base, 64), 64)])

    pltpu.sync_copy(mask_h.at[b], mrow)
    for k in range(4):
        st = pl.multiple_of(i0_base + k * L, L)
        msub[pl.ds(k * L, L)] = plsc.load_gather(mrow, [q_v[pl.ds(st, L)]])
    pltpu.sync_copy(msub, omask.at[b, pl.ds(pl.multiple_of(i0_base, 64), 64)])

    def ichunk(cc, _):
        i0 = pl.multiple_of(i0_base + cc * L, L)
        ridx_v[...] = q_v[pl.ds(i0, L)] + b * N
        pltpu.async_copy(ab_flat.at[ridx_v], rows_v, sem).wait()
        for il in range(L):
            row = rows_v.at[il]
            for jc in range(M // L):
                cols = q3_v[pl.ds(jc * L, L)]
                jvec = lanes + jc * L
                x = plsc.load_gather(row, [cols])
                y = plsc.load_gather(row, [cols + 1])
                z = plsc.load_gather(row, [cols + 2])
                plsc.store_scatter(stag, [jvec, jnp.full((L,), il * D, jnp.int32)], x)
                plsc.store_scatter(stag, [jvec, jnp.full((L,), il * D + 1, jnp.int32)], y)
                plsc.store_scatter(stag, [jvec, jnp.full((L,), il * D + 2, jnp.int32)], z)
        pltpu.sync_copy(stag, oab.at[b, :, pl.ds(i0 * D, L * D)])
        return 0

    lax.fori_loop(0, 4, ichunk, 0)


def _mesh():
    return plsc.VectorSubcoreMesh(core_axis_name="c", subcore_axis_name="s",
                                  num_cores=2, num_subcores=16)


@jax.jit
def _fps_call(ab_flat, f0):
    return pl.kernel(
        _fps_body,
        out_type=jax.ShapeDtypeStruct((BS, M), jnp.int32),
        mesh=_mesh(),
        compiler_params=pltpu.CompilerParams(needs_layout_passes=False, use_tc_tiling_on_sc=False),
        scratch_types=[
            pltpu.VMEM((N * D,), jnp.float32),
            pltpu.VMEM((N,), jnp.float32),
            pltpu.VMEM((M,), jnp.int32),
            pltpu.VMEM((L,), jnp.int32),
        ],
    )(ab_flat, f0)


@jax.jit
def _gather_call(ab3, vals_flat, mask_i32, q):
    return pl.kernel(
        _gather_body,
        out_type=(
            jax.ShapeDtypeStruct((BS, M, M * D), jnp.float32),
            jax.ShapeDtypeStruct((BS, M, C), jnp.float32),
            jax.ShapeDtypeStruct((BS, M), jnp.int32),
        ),
        mesh=_mesh(),
        compiler_params=pltpu.CompilerParams(needs_layout_passes=False, use_tc_tiling_on_sc=False),
        scratch_types=[
            pltpu.VMEM((M,), jnp.int32),
            pltpu.VMEM((M,), jnp.int32),
            pltpu.VMEM((L,), jnp.int32),
            pltpu.VMEM((L, N * D), jnp.float32),
            pltpu.VMEM((M, L * D), jnp.float32),
            pltpu.VMEM((64,), jnp.int32),
            pltpu.VMEM((64, C), jnp.float32),
            pltpu.VMEM((N,), jnp.int32),
            pltpu.VMEM((64,), jnp.int32),
            pltpu.SemaphoreType.DMA,
        ],
    )(ab3, vals_flat, mask_i32, q)


def kernel(ab_pairs, vals, mask):
    bs, n = mask.shape
    a = jax.random.randint(jax.random.key(42), (bs,), 0, n)
    msum = mask.sum(-1).astype(jnp.int32)
    k = a.astype(jnp.int32) % msum
    cum = jnp.cumsum(mask.astype(jnp.int32), axis=1)
    farthest0 = jnp.argmax(cum == (k[:, None] + 1), axis=1).astype(jnp.int32)
    f0 = jnp.zeros((L,), jnp.int32).at[:bs].set(farthest0)

    ab_flat = ab_pairs.reshape(bs * n, n * D)
    q = _fps_call(ab_flat, f0)
    oab, ovals, omask = _gather_call(
        ab_flat, vals.reshape(bs * n, C), mask.astype(jnp.int32), q)
    return oab.reshape(bs, M, M, D), ovals, omask.astype(bool)


# speedup vs baseline: 2.9305x; 2.0042x over previous
import jax
import jax.numpy as jnp
from jax import lax
from jax.experimental import pallas as pl
from jax.experimental.pallas import tpu as pltpu
from jax.experimental.pallas import tpu_sc as plsc

BS, N, D, C = 8, 1024, 3, 128
M = 256
L = 16
NCH = N // L


def _wid():
    return lax.axis_index("s") * 2 + lax.axis_index("c")


def _fps_body(abt_ref, f0_ref, q_ref, rowbuf, dist_ref, chosen_ref, f0_v,
              sem):
    w = _wid()

    @pl.when(w < BS)
    def _():
        b = w
        lanes = lax.iota(jnp.int32, L)
        pltpu.sync_copy(f0_ref, f0_v)
        f0 = jnp.max(jnp.where(lanes == b, f0_v[...].astype(jnp.float32),
                               0.0)).astype(jnp.int32)

        big = jnp.full((L,), 1e16, jnp.float32)
        for k in range(NCH):
            dist_ref[pl.ds(k * L, L)] = big

        def outer(o, f):
            def inner(t, carry):
                f, ch = carry
                ch = jnp.where(lanes == t, f, ch)
                ft = f // 8
                fr = f - ft * 8
                cps = [pltpu.async_copy(
                    abt_ref.at[b, c, ft, :, pl.ds(fr, 1), :],
                    rowbuf.at[c], sem) for c in range(D)]
                for cp in cps:
                    cp.wait()
                bestv = jnp.full((L,), -1.0, jnp.float32)
                besti = jnp.zeros((L,), jnp.int32)
                for k in range(NCH):
                    jt, jo = k // 8, (k % 8) * L
                    x = rowbuf[0, jt, 0, pl.ds(jo, L)]
                    y = rowbuf[1, jt, 0, pl.ds(jo, L)]
                    z = rowbuf[2, jt, 0, pl.ds(jo, L)]
                    dn = x * x + y * y + z * z
                    dm = jnp.minimum(dn, dist_ref[pl.ds(k * L, L)])
                    dist_ref[pl.ds(k * L, L)] = dm
                    upd = dm > bestv
                    bestv = jnp.where(upd, dm, bestv)
                    besti = jnp.where(upd, lanes + (k * L), besti)
                gm = jnp.max(bestv)
                cand = jnp.where(bestv == gm, besti.astype(jnp.float32), 1e9)
                return jnp.min(cand).astype(jnp.int32), ch

            f, ch = lax.fori_loop(0, L, inner, (f, jnp.zeros((L,), jnp.int32)))
            chosen_ref[pl.ds(pl.multiple_of(o * L, L), L)] = ch
            return f

        lax.fori_loop(0, M // L, outer, f0)
        pltpu.sync_copy(chosen_ref, q_ref.at[b])


def _gather_body(abt_ref, vals_h, mask_h, q_h, oab, ovals, omask,
                 q_v, qt_v, ql_v, rows_v, stag, vidx_v, vrow,
                 mrow, msub, sem):
    w = _wid()
    b = w // 4
    i0_base = (w % 4) * 64
    lanes = lax.iota(jnp.int32, L)

    pltpu.sync_copy(q_h.at[b], q_v)
    for k in range(M // L):
        qk = q_v[pl.ds(k * L, L)]
        qt_v[pl.ds(k * L, L)] = qk // 128
        ql_v[pl.ds(k * L, L)] = qk - (qk // 128) * 128

    for k in range(4):
        st = pl.multiple_of(i0_base + k * L, L)
        vidx_v[pl.ds(k * L, L)] = q_v[pl.ds(st, L)] + b * N
    pltpu.async_copy(vals_h.at[vidx_v], vrow, sem).wait()
    pltpu.sync_copy(vrow, ovals.at[b, pl.ds(pl.multiple_of(i0_base, 64), 64)])

    pltpu.sync_copy(mask_h.at[b], mrow)
    for k in range(4):
        st = pl.multiple_of(i0_base + k * L, L)
        msub[pl.ds(k * L, L)] = plsc.load_gather(mrow, [q_v[pl.ds(st, L)]])
    pltpu.sync_copy(msub, omask.at[b, pl.ds(pl.multiple_of(i0_base, 64), 64)])

    def ichunk(cc, _):
        i0 = pl.multiple_of(i0_base + cc * L, L)
        qchunk = q_v[pl.ds(i0, L)].astype(jnp.float32)
        cps = []
        for il in range(L):
            qi = jnp.max(jnp.where(lanes == il, qchunk, 0.0)).astype(jnp.int32)
            qft = qi // 8
            qfr = qi - qft * 8
            for c in range(D):
                cps.append(pltpu.async_copy(
                    abt_ref.at[b, c, qft, :, pl.ds(qfr, 1), :],
                    rows_v.at[c * L + il], sem))
        for cp in cps:
            cp.wait()
        for il in range(L):
            for jc in range(M // L):
                qt = qt_v[pl.ds(jc * L, L)]
                ql = ql_v[pl.ds(jc * L, L)]
                jvec = lanes + jc * L
                c0v = jnp.zeros((L,), jnp.int32)
                x = plsc.load_gather(rows_v, [jnp.full((L,), il, jnp.int32), qt, c0v, ql])
                y = plsc.load_gather(rows_v, [jnp.full((L,), L + il, jnp.int32), qt, c0v, ql])
                z = plsc.load_gather(rows_v, [jnp.full((L,), 2 * L + il, jnp.int32), qt, c0v, ql])
                plsc.store_scatter(stag, [jvec, jnp.full((L,), il * D, jnp.int32)], x)
                plsc.store_scatter(stag, [jvec, jnp.full((L,), il * D + 1, jnp.int32)], y)
                plsc.store_scatter(stag, [jvec, jnp.full((L,), il * D + 2, jnp.int32)], z)
        pltpu.sync_copy(stag, oab.at[b, :, pl.ds(i0 * D, L * D)])
        return 0

    lax.fori_loop(0, 4, ichunk, 0)


def _mesh():
    return plsc.VectorSubcoreMesh(core_axis_name="c", subcore_axis_name="s",
                                  num_cores=2, num_subcores=16)


@jax.jit
def _fps_call(abt, f0):
    return pl.kernel(
        _fps_body,
        out_type=jax.ShapeDtypeStruct((BS, M), jnp.int32),
        mesh=_mesh(),
        compiler_params=pltpu.CompilerParams(needs_layout_passes=False,
                                             use_tc_tiling_on_sc=False),
        scratch_types=[
            pltpu.VMEM((D, 8, 1, 128), jnp.float32),
            pltpu.VMEM((N,), jnp.float32),
            pltpu.VMEM((M,), jnp.int32),
            pltpu.VMEM((L,), jnp.int32),
            pltpu.SemaphoreType.DMA,
        ],
    )(abt, f0)


@jax.jit
def _gather_call(abt, vals_flat, mask_i32, q):
    return pl.kernel(
        _gather_body,
        out_type=(
            jax.ShapeDtypeStruct((BS, M, M * D), jnp.float32),
            jax.ShapeDtypeStruct((BS, M, C), jnp.float32),
            jax.ShapeDtypeStruct((BS, M), jnp.int32),
        ),
        mesh=_mesh(),
        compiler_params=pltpu.CompilerParams(needs_layout_passes=False,
                                             use_tc_tiling_on_sc=False),
        scratch_types=[
            pltpu.VMEM((M,), jnp.int32),
            pltpu.VMEM((M,), jnp.int32),
            pltpu.VMEM((M,), jnp.int32),
            pltpu.VMEM((D * L, 8, 1, 128), jnp.float32),
            pltpu.VMEM((M, L * D), jnp.float32),
            pltpu.VMEM((64,), jnp.int32),
            pltpu.VMEM((64, C), jnp.float32),
            pltpu.VMEM((N,), jnp.int32),
            pltpu.VMEM((64,), jnp.int32),
            pltpu.SemaphoreType.DMA,
        ],
    )(abt, vals_flat, mask_i32, q)


def kernel(ab_pairs, vals, mask):
    bs, n = mask.shape
    a = jax.random.randint(jax.random.key(42), (bs,), 0, n)
    msum = mask.sum(-1).astype(jnp.int32)
    k = a.astype(jnp.int32) % msum
    cum = jnp.cumsum(mask.astype(jnp.int32), axis=1)
    farthest0 = jnp.argmax(cum == (k[:, None] + 1), axis=1).astype(jnp.int32)
    f0 = jnp.zeros((L,), jnp.int32).at[:bs].set(farthest0)

    abt = ab_pairs.reshape(bs, n // 8, 8, 8, 128, D).transpose(0, 5, 1, 3, 2, 4)
    q = _fps_call(abt, f0)
    oab, ovals, omask = _gather_call(
        abt, vals.reshape(bs * n, C), mask.astype(jnp.int32), q)
    return oab.reshape(bs, M, M, D), ovals, omask.astype(bool)


# speedup vs baseline: 2.9837x; 1.0182x over previous
import jax
import jax.numpy as jnp
from jax import lax
from jax.experimental import pallas as pl
from jax.experimental.pallas import tpu as pltpu
from jax.experimental.pallas import tpu_sc as plsc

BS, N, D, C = 8, 1024, 3, 128
M = 256
L = 16
NCH = N // L


def _wid():
    return lax.axis_index("s") * 2 + lax.axis_index("c")


def _fps_body(abt_ref, f0_ref, q_ref, rowbuf, dist_ref, chosen_ref, f0_v,
              sem):
    w = _wid()

    @pl.when(w < BS)
    def _():
        b = w
        lanes = lax.iota(jnp.int32, L)
        pltpu.sync_copy(f0_ref, f0_v)
        f0 = jnp.max(jnp.where(lanes == b, f0_v[...].astype(jnp.float32),
                               0.0)).astype(jnp.int32)

        big = jnp.full((L,), 1e16, jnp.float32)
        for k in range(NCH):
            dist_ref[pl.ds(k * L, L)] = big

        def outer(o, f):
            def inner(t, carry):
                f, ch = carry
                ch = jnp.where(lanes == t, f, ch)
                ft = f // 8
                fr = f - ft * 8
                pltpu.async_copy(
                    abt_ref.at[b, :, ft, :, pl.ds(fr, 1), :],
                    rowbuf, sem).wait()
                bestv = jnp.full((L,), -1.0, jnp.float32)
                besti = jnp.zeros((L,), jnp.int32)
                for k in range(NCH):
                    jt, jo = k // 8, (k % 8) * L
                    x = rowbuf[0, jt, 0, pl.ds(jo, L)]
                    y = rowbuf[1, jt, 0, pl.ds(jo, L)]
                    z = rowbuf[2, jt, 0, pl.ds(jo, L)]
                    dn = x * x + y * y + z * z
                    dm = jnp.minimum(dn, dist_ref[pl.ds(k * L, L)])
                    dist_ref[pl.ds(k * L, L)] = dm
                    upd = dm > bestv
                    bestv = jnp.where(upd, dm, bestv)
                    besti = jnp.where(upd, lanes + (k * L), besti)
                gm = jnp.max(bestv)
                cand = jnp.where(bestv == gm, besti.astype(jnp.float32), 1e9)
                return jnp.min(cand).astype(jnp.int32), ch

            f, ch = lax.fori_loop(0, L, inner, (f, jnp.zeros((L,), jnp.int32)))
            chosen_ref[pl.ds(pl.multiple_of(o * L, L), L)] = ch
            return f

        lax.fori_loop(0, M // L, outer, f0)
        pltpu.sync_copy(chosen_ref, q_ref.at[b])


def _gather_body(abt_ref, vals_h, mask_h, q_h, oab, ovals, omask,
                 q_v, qt_v, ql_v, rows_v, stag, vidx_v, vrow,
                 mrow, msub, sem):
    w = _wid()
    b = w // 4
    i0_base = (w % 4) * 64
    lanes = lax.iota(jnp.int32, L)

    pltpu.sync_copy(q_h.at[b], q_v)
    for k in range(M // L):
        qk = q_v[pl.ds(k * L, L)]
        qt_v[pl.ds(k * L, L)] = qk // 128
        ql_v[pl.ds(k * L, L)] = qk - (qk // 128) * 128

    for k in range(4):
        st = pl.multiple_of(i0_base + k * L, L)
        vidx_v[pl.ds(k * L, L)] = q_v[pl.ds(st, L)] + b * N
    pltpu.async_copy(vals_h.at[vidx_v], vrow, sem).wait()
    pltpu.sync_copy(vrow, ovals.at[b, pl.ds(pl.multiple_of(i0_base, 64), 64)])

    pltpu.sync_copy(mask_h.at[b], mrow)
    for k in range(4):
        st = pl.multiple_of(i0_base + k * L, L)
        msub[pl.ds(k * L, L)] = plsc.load_gather(mrow, [q_v[pl.ds(st, L)]])
    pltpu.sync_copy(msub, omask.at[b, pl.ds(pl.multiple_of(i0_base, 64), 64)])

    def ichunk(cc, _):
        i0 = pl.multiple_of(i0_base + cc * L, L)
        qchunk = q_v[pl.ds(i0, L)].astype(jnp.float32)
        cps = []
        for il in range(L):
            qi = jnp.max(jnp.where(lanes == il, qchunk, 0.0)).astype(jnp.int32)
            qft = qi // 8
            qfr = qi - qft * 8
            cps.append(pltpu.async_copy(
                abt_ref.at[b, :, qft, :, pl.ds(qfr, 1), :],
                rows_v.at[il], sem))
        for cp in cps:
            cp.wait()
        for il in range(L):
            for jc in range(M // L):
                qt = qt_v[pl.ds(jc * L, L)]
                ql = ql_v[pl.ds(jc * L, L)]
                jvec = lanes + jc * L
                c0v = jnp.zeros((L,), jnp.int32)
                ilv = jnp.full((L,), il, jnp.int32)
                x = plsc.load_gather(rows_v, [ilv, c0v, qt, c0v, ql])
                y = plsc.load_gather(rows_v, [ilv, c0v + 1, qt, c0v, ql])
                z = plsc.load_gather(rows_v, [ilv, c0v + 2, qt, c0v, ql])
                plsc.store_scatter(stag, [jvec, jnp.full((L,), il * D, jnp.int32)], x)
                plsc.store_scatter(stag, [jvec, jnp.full((L,), il * D + 1, jnp.int32)], y)
                plsc.store_scatter(stag, [jvec, jnp.full((L,), il * D + 2, jnp.int32)], z)
        pltpu.sync_copy(stag, oab.at[b, :, pl.ds(i0 * D, L * D)])
        return 0

    lax.fori_loop(0, 4, ichunk, 0)


def _mesh():
    return plsc.VectorSubcoreMesh(core_axis_name="c", subcore_axis_name="s",
                                  num_cores=2, num_subcores=16)


@jax.jit
def _fps_call(abt, f0):
    return pl.kernel(
        _fps_body,
        out_type=jax.ShapeDtypeStruct((BS, M), jnp.int32),
        mesh=_mesh(),
        compiler_params=pltpu.CompilerParams(needs_layout_passes=False,
                                             use_tc_tiling_on_sc=False),
        scratch_types=[
            pltpu.VMEM((D, 8, 1, 128), jnp.float32),
            pltpu.VMEM((N,), jnp.float32),
            pltpu.VMEM((M,), jnp.int32),
            pltpu.VMEM((L,), jnp.int32),
            pltpu.SemaphoreType.DMA,
        ],
    )(abt, f0)


@jax.jit
def _gather_call(abt, vals_flat, mask_i32, q):
    return pl.kernel(
        _gather_body,
        out_type=(
            jax.ShapeDtypeStruct((BS, M, M * D), jnp.float32),
            jax.ShapeDtypeStruct((BS, M, C), jnp.float32),
            jax.ShapeDtypeStruct((BS, M), jnp.int32),
        ),
        mesh=_mesh(),
        compiler_params=pltpu.CompilerParams(needs_layout_passes=False,
                                             use_tc_tiling_on_sc=False),
        scratch_types=[
            pltpu.VMEM((M,), jnp.int32),
            pltpu.VMEM((M,), jnp.int32),
            pltpu.VMEM((M,), jnp.int32),
            pltpu.VMEM((L, D, 8, 1, 128), jnp.float32),
            pltpu.VMEM((M, L * D), jnp.float32),
            pltpu.VMEM((64,), jnp.int32),
            pltpu.VMEM((64, C), jnp.float32),
            pltpu.VMEM((N,), jnp.int32),
            pltpu.VMEM((64,), jnp.int32),
            pltpu.SemaphoreType.DMA,
        ],
    )(abt, vals_flat, mask_i32, q)


def kernel(ab_pairs, vals, mask):
    bs, n = mask.shape
    a = jax.random.randint(jax.random.key(42), (bs,), 0, n)
    msum = mask.sum(-1).astype(jnp.int32)
    k = a.astype(jnp.int32) % msum
    cum = jnp.cumsum(mask.astype(jnp.int32), axis=1)
    farthest0 = jnp.argmax(cum == (k[:, None] + 1), axis=1).astype(jnp.int32)
    f0 = jnp.zeros((L,), jnp.int32).at[:bs].set(farthest0)

    abt = ab_pairs.reshape(bs, n // 8, 8, 8, 128, D).transpose(0, 5, 1, 3, 2, 4)
    q = _fps_call(abt, f0)
    oab, ovals, omask = _gather_call(
        abt, vals.reshape(bs * n, C), mask.astype(jnp.int32), q)
    return oab.reshape(bs, M, M, D), ovals, omask.astype(bool)


# speedup vs baseline: 3.3612x; 1.1265x over previous
import jax
import jax.numpy as jnp
from jax import lax
from jax.experimental import pallas as pl
from jax.experimental.pallas import tpu as pltpu
from jax.experimental.pallas import tpu_sc as plsc

BS, N, D, C = 8, 1024, 3, 128
M = 256
L = 16
NCH = N // L


def _wid():
    return lax.axis_index("s") * 2 + lax.axis_index("c")


def _fps_body(abt_ref, f0_ref, q_ref, rowbuf, dist_ref, chosen_ref, f0_v,
              sem, sem2):
    w = _wid()

    @pl.when(w < BS)
    def _():
        b = w
        lanes = lax.iota(jnp.int32, L)
        pltpu.sync_copy(f0_ref, f0_v)
        f0 = jnp.max(jnp.where(lanes == b, f0_v[...].astype(jnp.float32),
                               0.0)).astype(jnp.int32)

        big = jnp.full((L,), 1e16, jnp.float32)
        for k in range(NCH):
            dist_ref[pl.ds(k * L, L)] = big

        def outer(o, f):
            def inner(t, carry):
                f, ch = carry
                ch = jnp.where(lanes == t, f, ch)
                ft = f // 8
                fr = f - ft * 8
                cp1 = pltpu.async_copy(
                    abt_ref.at[b, :, ft, pl.ds(0, 4), pl.ds(fr, 1), :],
                    rowbuf.at[:, pl.ds(0, 4)], sem)
                cp2 = pltpu.async_copy(
                    abt_ref.at[b, :, ft, pl.ds(4, 4), pl.ds(fr, 1), :],
                    rowbuf.at[:, pl.ds(4, 4)], sem2)
                bestv = jnp.full((L,), -1.0, jnp.float32)
                besti = jnp.zeros((L,), jnp.int32)
                cp1.wait()
                for k in range(NCH // 2):
                    jt, jo = k // 8, (k % 8) * L
                    x = rowbuf[0, jt, 0, pl.ds(jo, L)]
                    y = rowbuf[1, jt, 0, pl.ds(jo, L)]
                    z = rowbuf[2, jt, 0, pl.ds(jo, L)]
                    dn = x * x + y * y + z * z
                    dm = jnp.minimum(dn, dist_ref[pl.ds(k * L, L)])
                    dist_ref[pl.ds(k * L, L)] = dm
                    upd = dm > bestv
                    bestv = jnp.where(upd, dm, bestv)
                    besti = jnp.where(upd, lanes + (k * L), besti)
                cp2.wait()
                for k in range(NCH // 2, NCH):
                    jt, jo = k // 8, (k % 8) * L
                    x = rowbuf[0, jt, 0, pl.ds(jo, L)]
                    y = rowbuf[1, jt, 0, pl.ds(jo, L)]
                    z = rowbuf[2, jt, 0, pl.ds(jo, L)]
                    dn = x * x + y * y + z * z
                    dm = jnp.minimum(dn, dist_ref[pl.ds(k * L, L)])
                    dist_ref[pl.ds(k * L, L)] = dm
                    upd = dm > bestv
                    bestv = jnp.where(upd, dm, bestv)
                    besti = jnp.where(upd, lanes + (k * L), besti)
                gm = jnp.max(bestv)
                cand = jnp.where(bestv == gm, besti.astype(jnp.float32), 1e9)
                return jnp.min(cand).astype(jnp.int32), ch

            f, ch = lax.fori_loop(0, L, inner, (f, jnp.zeros((L,), jnp.int32)))
            chosen_ref[pl.ds(pl.multiple_of(o * L, L), L)] = ch
            return f

        lax.fori_loop(0, M // L, outer, f0)
        pltpu.sync_copy(chosen_ref, q_ref.at[b])


def _gather_body(abt_ref, vals_h, mask_h, q_h, oab, ovals, omask,
                 q_v, qt_v, ql_v, rows_v, rows_v2, stag, vidx_v, vrow,
                 mrow, msub, sem, sem2):
    w = _wid()
    b = w // 4
    i0_base = (w % 4) * 64
    lanes = lax.iota(jnp.int32, L)

    pltpu.sync_copy(q_h.at[b], q_v)
    for k in range(M // L):
        qk = q_v[pl.ds(k * L, L)]
        qt_v[pl.ds(k * L, L)] = qk // 128
        ql_v[pl.ds(k * L, L)] = qk - (qk // 128) * 128

    for k in range(4):
        st = pl.multiple_of(i0_base + k * L, L)
        vidx_v[pl.ds(k * L, L)] = q_v[pl.ds(st, L)] + b * N
    pltpu.async_copy(vals_h.at[vidx_v], vrow, sem).wait()
    pltpu.sync_copy(vrow, ovals.at[b, pl.ds(pl.multiple_of(i0_base, 64), 64)])

    pltpu.sync_copy(mask_h.at[b], mrow)
    for k in range(4):
        st = pl.multiple_of(i0_base + k * L, L)
        msub[pl.ds(k * L, L)] = plsc.load_gather(mrow, [q_v[pl.ds(st, L)]])
    pltpu.sync_copy(msub, omask.at[b, pl.ds(pl.multiple_of(i0_base, 64), 64)])

    def issue_rows(s8, buf, s):
        qchunk = q_v[pl.ds(pl.multiple_of(i0_base + (s8 // 2) * L, L),
                           L)].astype(jnp.float32)
        cps = []
        for r in range(8):
            il = (s8 % 2) * 8 + r
            qi = jnp.max(jnp.where(lanes == il, qchunk, 0.0)).astype(jnp.int32)
            qft = qi // 8
            qfr = qi - qft * 8
            cps.append(pltpu.async_copy(
                abt_ref.at[b, :, qft, :, pl.ds(qfr, 1), :],
                buf.at[r], s))
        return cps

    bufs = (rows_v, rows_v2)
    sems = (sem, sem2)
    pend = issue_rows(0, bufs[0], sems[0])
    for s8 in range(8):
        if s8 < 7:
            nxt = issue_rows(s8 + 1, bufs[(s8 + 1) % 2], sems[(s8 + 1) % 2])
        for cp in pend:
            cp.wait()
        rv = bufs[s8 % 2]

        def row_body(r, _):
            il = (s8 % 2) * 8 + r
            c0v = jnp.zeros((L,), jnp.int32)
            rv_i = jnp.broadcast_to(r, (L,)).astype(jnp.int32)
            ilD = jnp.broadcast_to(il * D, (L,)).astype(jnp.int32)
            for jc in range(M // L):
                qt = qt_v[pl.ds(jc * L, L)]
                ql = ql_v[pl.ds(jc * L, L)]
                jvec = lanes + jc * L
                x = plsc.load_gather(rv, [rv_i, c0v, qt, c0v, ql])
                y = plsc.load_gather(rv, [rv_i, c0v + 1, qt, c0v, ql])
                z = plsc.load_gather(rv, [rv_i, c0v + 2, qt, c0v, ql])
                plsc.store_scatter(stag, [jvec, ilD], x)
                plsc.store_scatter(stag, [jvec, ilD + 1], y)
                plsc.store_scatter(stag, [jvec, ilD + 2], z)
            return 0

        lax.fori_loop(0, 8, row_body, 0)
        if s8 % 2 == 1:
            i0 = pl.multiple_of(i0_base + (s8 // 2) * L, L)
            pltpu.sync_copy(stag, oab.at[b, :, pl.ds(i0 * D, L * D)])
        if s8 < 7:
            pend = nxt


def _mesh():
    return plsc.VectorSubcoreMesh(core_axis_name="c", subcore_axis_name="s",
                                  num_cores=2, num_subcores=16)


@jax.jit
def _fps_call(abt, f0):
    return pl.kernel(
        _fps_body,
        out_type=jax.ShapeDtypeStruct((BS, M), jnp.int32),
        mesh=_mesh(),
        compiler_params=pltpu.CompilerParams(needs_layout_passes=False,
                                             use_tc_tiling_on_sc=False),
        scratch_types=[
            pltpu.VMEM((D, 8, 1, 128), jnp.float32),
            pltpu.VMEM((N,), jnp.float32),
            pltpu.VMEM((M,), jnp.int32),
            pltpu.VMEM((L,), jnp.int32),
            pltpu.SemaphoreType.DMA,
            pltpu.SemaphoreType.DMA,
        ],
    )(abt, f0)


@jax.jit
def _gather_call(abt, vals_flat, mask_i32, q):
    return pl.kernel(
        _gather_body,
        out_type=(
            jax.ShapeDtypeStruct((BS, M, M * D), jnp.float32),
            jax.ShapeDtypeStruct((BS, M, C), jnp.float32),
            jax.ShapeDtypeStruct((BS, M), jnp.int32),
        ),
        mesh=_mesh(),
        compiler_params=pltpu.CompilerParams(needs_layout_passes=False,
                                             use_tc_tiling_on_sc=False),
        scratch_types=[
            pltpu.VMEM((M,), jnp.int32),
            pltpu.VMEM((M,), jnp.int32),
            pltpu.VMEM((M,), jnp.int32),
            pltpu.VMEM((8, D, 8, 1, 128), jnp.float32),
            pltpu.VMEM((8, D, 8, 1, 128), jnp.float32),
            pltpu.VMEM((M, L * D), jnp.float32),
            pltpu.VMEM((64,), jnp.int32),
            pltpu.VMEM((64, C), jnp.float32),
            pltpu.VMEM((N,), jnp.int32),
            pltpu.VMEM((64,), jnp.int32),
            pltpu.SemaphoreType.DMA,
            pltpu.SemaphoreType.DMA,
        ],
    )(abt, vals_flat, mask_i32, q)


def kernel(ab_pairs, vals, mask):
    bs, n = mask.shape
    a = jax.random.randint(jax.random.key(42), (bs,), 0, n)
    msum = mask.sum(-1).astype(jnp.int32)
    k = a.astype(jnp.int32) % msum
    cum = jnp.cumsum(mask.astype(jnp.int32), axis=1)
    farthest0 = jnp.argmax(cum == (k[:, None] + 1), axis=1).astype(jnp.int32)
    f0 = jnp.zeros((L,), jnp.int32).at[:bs].set(farthest0)

    abt = ab_pairs.reshape(bs, n // 8, 8, 8, 128, D).transpose(0, 5, 1, 3, 2, 4)
    q = _fps_call(abt, f0)
    oab, ovals, omask = _gather_call(
        abt, vals.reshape(bs * n, C), mask.astype(jnp.int32), q)
    return oab.reshape(bs, M, M, D), ovals, omask.astype(bool)


# speedup vs baseline: 3.4505x; 1.0266x over previous
import jax
import jax.numpy as jnp
from jax import lax
from jax.experimental import pallas as pl
from jax.experimental.pallas import tpu as pltpu
from jax.experimental.pallas import tpu_sc as plsc

BS, N, D, C = 8, 1024, 3, 128
M = 256
L = 16
NCH = N // L


def _wid():
    return lax.axis_index("s") * 2 + lax.axis_index("c")


def _fps_body(abt_ref, f0_ref, q_ref, rowbuf, dist_ref, chosen_ref, f0_v,
              sem, sem2):
    w = _wid()

    @pl.when(w < BS)
    def _():
        b = w
        lanes = lax.iota(jnp.int32, L)
        pltpu.sync_copy(f0_ref, f0_v)
        f0 = jnp.max(jnp.where(lanes == b, f0_v[...].astype(jnp.float32),
                               0.0)).astype(jnp.int32)

        big = jnp.full((L,), 1e16, jnp.float32)
        for k in range(NCH):
            dist_ref[pl.ds(k * L, L)] = big

        def outer(o, f):
            def inner(t, carry):
                f, ch = carry
                ch = jnp.where(lanes == t, f, ch)
                ft = f // 8
                fr = f - ft * 8
                cp1 = pltpu.async_copy(
                    abt_ref.at[b, :, ft, pl.ds(0, 4), pl.ds(fr, 1), :],
                    rowbuf.at[:, pl.ds(0, 4)], sem)
                cp2 = pltpu.async_copy(
                    abt_ref.at[b, :, ft, pl.ds(4, 4), pl.ds(fr, 1), :],
                    rowbuf.at[:, pl.ds(4, 4)], sem2)
                bestv = jnp.full((L,), -1.0, jnp.float32)
                besti = jnp.zeros((L,), jnp.int32)
                cp1.wait()
                for k in range(NCH // 2):
                    jt, jo = k // 8, (k % 8) * L
                    x = rowbuf[0, jt, 0, pl.ds(jo, L)]
                    y = rowbuf[1, jt, 0, pl.ds(jo, L)]
                    z = rowbuf[2, jt, 0, pl.ds(jo, L)]
                    dn = x * x + y * y + z * z
                    dm = jnp.minimum(dn, dist_ref[pl.ds(k * L, L)])
                    dist_ref[pl.ds(k * L, L)] = dm
                    upd = dm > bestv
                    bestv = jnp.where(upd, dm, bestv)
                    besti = jnp.where(upd, lanes + (k * L), besti)
                cp2.wait()
                for k in range(NCH // 2, NCH):
                    jt, jo = k // 8, (k % 8) * L
                    x = rowbuf[0, jt, 0, pl.ds(jo, L)]
                    y = rowbuf[1, jt, 0, pl.ds(jo, L)]
                    z = rowbuf[2, jt, 0, pl.ds(jo, L)]
                    dn = x * x + y * y + z * z
                    dm = jnp.minimum(dn, dist_ref[pl.ds(k * L, L)])
                    dist_ref[pl.ds(k * L, L)] = dm
                    upd = dm > bestv
                    bestv = jnp.where(upd, dm, bestv)
                    besti = jnp.where(upd, lanes + (k * L), besti)
                gm = jnp.max(bestv)
                cand = jnp.where(bestv == gm, besti.astype(jnp.float32), 1e9)
                return jnp.min(cand).astype(jnp.int32), ch

            f, ch = lax.fori_loop(0, L, inner, (f, jnp.zeros((L,), jnp.int32)))
            chosen_ref[pl.ds(pl.multiple_of(o * L, L), L)] = ch
            return f

        lax.fori_loop(0, M // L, outer, f0)
        pltpu.sync_copy(chosen_ref, q_ref.at[b])


def _gather_body(abt_ref, vals_h, mask_h, q_h, oab, ovals, omask,
                 q_v, qt_v, ql_v, rows_v, rows_v2, stag3, vidx_v, vrow,
                 mrow, msub, sem, sem2):
    w = _wid()
    b = w // 4
    i0_base = (w % 4) * 64
    lanes = lax.iota(jnp.int32, L)

    pltpu.sync_copy(q_h.at[b], q_v)
    for k in range(M // L):
        qk = q_v[pl.ds(k * L, L)]
        qt_v[pl.ds(k * L, L)] = qk // 128
        ql_v[pl.ds(k * L, L)] = qk - (qk // 128) * 128
    jtb = lanes // 8
    jrv = lanes - jtb * 8

    for k in range(4):
        st = pl.multiple_of(i0_base + k * L, L)
        vidx_v[pl.ds(k * L, L)] = q_v[pl.ds(st, L)] + b * N
    pltpu.async_copy(vals_h.at[vidx_v], vrow, sem).wait()
    pltpu.sync_copy(vrow, ovals.at[b, pl.ds(pl.multiple_of(i0_base, 64), 64)])

    pltpu.sync_copy(mask_h.at[b], mrow)
    for k in range(4):
        st = pl.multiple_of(i0_base + k * L, L)
        msub[pl.ds(k * L, L)] = plsc.load_gather(mrow, [q_v[pl.ds(st, L)]])
    pltpu.sync_copy(msub, omask.at[b, pl.ds(pl.multiple_of(i0_base, 64), 64)])

    def issue_rows(s8, buf, s):
        qchunk = q_v[pl.ds(pl.multiple_of(i0_base + (s8 // 2) * L, L),
                           L)].astype(jnp.float32)
        cps = []
        for r in range(8):
            il = (s8 % 2) * 8 + r
            qi = jnp.max(jnp.where(lanes == il, qchunk, 0.0)).astype(jnp.int32)
            qft = qi // 8
            qfr = qi - qft * 8
            cps.append(pltpu.async_copy(
                abt_ref.at[b, :, qft, :, pl.ds(qfr, 1), :],
                buf.at[r], s))
        return cps

    bufs = (rows_v, rows_v2)
    sems = (sem, sem2)
    pend = issue_rows(0, bufs[0], sems[0])
    for s8 in range(8):
        if s8 < 7:
            nxt = issue_rows(s8 + 1, bufs[(s8 + 1) % 2], sems[(s8 + 1) % 2])
        for cp in pend:
            cp.wait()
        rv = bufs[s8 % 2]

        def row_body(r, _):
            il = (s8 % 2) * 8 + r
            c0v = jnp.zeros((L,), jnp.int32)
            rv_i = jnp.broadcast_to(r, (L,)).astype(jnp.int32)
            ilv = jnp.broadcast_to(il, (L,)).astype(jnp.int32)
            for jc in range(M // L):
                qt = qt_v[pl.ds(jc * L, L)]
                ql = ql_v[pl.ds(jc * L, L)]
                jtv = jtb + jc * 2
                x = plsc.load_gather(rv, [rv_i, c0v, qt, c0v, ql])
                y = plsc.load_gather(rv, [rv_i, c0v + 1, qt, c0v, ql])
                z = plsc.load_gather(rv, [rv_i, c0v + 2, qt, c0v, ql])
                plsc.store_scatter(stag3, [c0v, jtv, jrv, ilv], x)
                plsc.store_scatter(stag3, [c0v + 1, jtv, jrv, ilv], y)
                plsc.store_scatter(stag3, [c0v + 2, jtv, jrv, ilv], z)
            return 0

        lax.fori_loop(0, 8, row_body, 0)
        if s8 % 2 == 1:
            i0 = pl.multiple_of(i0_base + (s8 // 2) * L, L)
            it = i0 // 128
            ir0 = pl.multiple_of(i0 - it * 128, L)
            pltpu.sync_copy(stag3, oab.at[b, :, :, it, :, pl.ds(ir0, L)])
        if s8 < 7:
            pend = nxt


def _mesh():
    return plsc.VectorSubcoreMesh(core_axis_name="c", subcore_axis_name="s",
                                  num_cores=2, num_subcores=16)


@jax.jit
def _fps_call(abt, f0):
    return pl.kernel(
        _fps_body,
        out_type=jax.ShapeDtypeStruct((BS, M), jnp.int32),
        mesh=_mesh(),
        compiler_params=pltpu.CompilerParams(needs_layout_passes=False,
                                             use_tc_tiling_on_sc=False),
        scratch_types=[
            pltpu.VMEM((D, 8, 1, 128), jnp.float32),
            pltpu.VMEM((N,), jnp.float32),
            pltpu.VMEM((M,), jnp.int32),
            pltpu.VMEM((L,), jnp.int32),
            pltpu.SemaphoreType.DMA,
            pltpu.SemaphoreType.DMA,
        ],
    )(abt, f0)


@jax.jit
def _gather_call(abt, vals_flat, mask_i32, q):
    return pl.kernel(
        _gather_body,
        out_type=(
            jax.ShapeDtypeStruct((BS, D, 32, 2, 8, 128), jnp.float32),
            jax.ShapeDtypeStruct((BS, M, C), jnp.float32),
            jax.ShapeDtypeStruct((BS, M), jnp.int32),
        ),
        mesh=_mesh(),
        compiler_params=pltpu.CompilerParams(needs_layout_passes=False,
                                             use_tc_tiling_on_sc=False),
        scratch_types=[
            pltpu.VMEM((M,), jnp.int32),
            pltpu.VMEM((M,), jnp.int32),
            pltpu.VMEM((M,), jnp.int32),
            pltpu.VMEM((8, D, 8, 1, 128), jnp.float32),
            pltpu.VMEM((8, D, 8, 1, 128), jnp.float32),
            pltpu.VMEM((D, 32, 8, L), jnp.float32),
            pltpu.VMEM((64,), jnp.int32),
            pltpu.VMEM((64, C), jnp.float32),
            pltpu.VMEM((N,), jnp.int32),
            pltpu.VMEM((64,), jnp.int32),
            pltpu.SemaphoreType.DMA,
            pltpu.SemaphoreType.DMA,
        ],
    )(abt, vals_flat, mask_i32, q)


def kernel(ab_pairs, vals, mask):
    bs, n = mask.shape
    a = jax.random.randint(jax.random.key(42), (bs,), 0, n)
    msum = mask.sum(-1).astype(jnp.int32)
    k = a.astype(jnp.int32) % msum
    cum = jnp.cumsum(mask.astype(jnp.int32), axis=1)
    farthest0 = jnp.argmax(cum == (k[:, None] + 1), axis=1).astype(jnp.int32)
    f0 = jnp.zeros((L,), jnp.int32).at[:bs].set(farthest0)

    abt = ab_pairs.reshape(bs, n // 8, 8, 8, 128, D).transpose(0, 5, 1, 3, 2, 4)
    q = _fps_call(abt, f0)
    oab6, ovals, omask = _gather_call(
        abt, vals.reshape(bs * n, C), mask.astype(jnp.int32), q)
    oab = oab6.transpose(0, 2, 4, 3, 5, 1).reshape(bs, M, M, D)
    return oab, ovals, omask.astype(bool)
